# baseline (device time: 181851 ns/iter reference)
import jax
import jax.numpy as jnp
from jax import lax
from jax.experimental import pallas as pl
from jax.experimental.pallas import tpu as pltpu

N_DEV = 8


def kernel(x, w_mat):
    m_global, k_shard = x.shape
    _, n = w_mat.shape
    m_per = m_global // N_DEV

    def body(x_ref, w_ref, out_ref, send_ref, recv_ref, send_sem, recv_sems):
        my = lax.axis_index("i")
        left = lax.rem(my + N_DEV - 1, N_DEV)
        right = lax.rem(my + 1, N_DEV)

        barrier = pltpu.get_barrier_semaphore()
        pl.semaphore_signal(barrier, inc=1, device_id=(left,),
                            device_id_type=pl.DeviceIdType.MESH)
        pl.semaphore_signal(barrier, inc=1, device_id=(right,),
                            device_id_type=pl.DeviceIdType.MESH)
        pl.semaphore_wait(barrier, 2)

        def local_chunk(c):
            xc = x_ref[pl.ds(c * m_per, m_per), :]
            return lax.dot_general(
                xc, w_ref[:, :],
                dimension_numbers=(((1,), (0,)), ((), ())),
                preferred_element_type=jnp.float32,
            )

        for s in range(N_DEV - 1):
            c_send = lax.rem(my + N_DEV - s - 1, N_DEV)
            lc = local_chunk(c_send)
            if s == 0:
                send_ref[:, :] = lc
            else:
                send_ref[:, :] = recv_ref[s - 1] + lc
            rdma = pltpu.make_async_remote_copy(
                src_ref=send_ref,
                dst_ref=recv_ref.at[s],
                send_sem=send_sem,
                recv_sem=recv_sems.at[s],
                device_id=(right,),
                device_id_type=pl.DeviceIdType.MESH,
            )
            rdma.start()
            rdma.wait()

        acc = recv_ref[N_DEV - 2] + local_chunk(my)
        out_ref[:, :] = jnp.maximum(acc, 0.0)

    return pl.pallas_call(
        body,
        out_shape=jax.ShapeDtypeStruct((m_per, n), jnp.float32),
        in_specs=[
            pl.BlockSpec(memory_space=pltpu.VMEM),
            pl.BlockSpec(memory_space=pltpu.VMEM),
        ],
        out_specs=pl.BlockSpec(memory_space=pltpu.VMEM),
        scratch_shapes=[
            pltpu.VMEM((m_per, n), jnp.float32),
            pltpu.VMEM((N_DEV - 1, m_per, n), jnp.float32),
            pltpu.SemaphoreType.DMA,
            pltpu.SemaphoreType.DMA((N_DEV - 1,)),
        ],
        compiler_params=pltpu.CompilerParams(collective_id=0),
    )(x, w_mat)


# device time: 104293 ns/iter; 1.7437x vs baseline; 1.7437x over previous
import jax
import jax.numpy as jnp
from jax import lax
from jax.experimental import pallas as pl
from jax.experimental.pallas import tpu as pltpu

N_DEV = 8


def kernel(x, w_mat):
    m_global, k_shard = x.shape
    _, n = w_mat.shape
    m_per = m_global // N_DEV
    nh = n // 2

    def body(x_ref, w_ref, out_ref,
             send_r, send_l, recv_r, recv_l,
             send_sems_r, send_sems_l, recv_sems_r, recv_sems_l):
        my = lax.axis_index("i")
        left = lax.rem(my + N_DEV - 1, N_DEV)
        right = lax.rem(my + 1, N_DEV)

        barrier = pltpu.get_barrier_semaphore()
        pl.semaphore_signal(barrier, inc=1, device_id=(left,),
                            device_id_type=pl.DeviceIdType.MESH)
        pl.semaphore_signal(barrier, inc=1, device_id=(right,),
                            device_id_type=pl.DeviceIdType.MESH)
        pl.semaphore_wait(barrier, 2)

        def lc_r(c):
            xc = x_ref[pl.ds(c * m_per, m_per), :]
            return lax.dot_general(
                xc, w_ref[:, :nh],
                dimension_numbers=(((1,), (0,)), ((), ())),
                preferred_element_type=jnp.float32,
            )

        def lc_l(c):
            xc = x_ref[pl.ds(c * m_per, m_per), :]
            return lax.dot_general(
                xc, w_ref[:, nh:],
                dimension_numbers=(((1,), (0,)), ((), ())),
                preferred_element_type=jnp.float32,
            )

        send_r[0, :, :] = lc_r(lax.rem(my + N_DEV - 1, N_DEV))
        send_l[0, :, :] = lc_l(lax.rem(my + 1, N_DEV))

        rdmas_r = []
        rdmas_l = []
        for s in range(N_DEV - 1):
            slot = s % 2
            r = pltpu.make_async_remote_copy(
                src_ref=send_r.at[slot],
                dst_ref=recv_r.at[s],
                send_sem=send_sems_r.at[slot],
                recv_sem=recv_sems_r.at[s],
                device_id=(right,),
                device_id_type=pl.DeviceIdType.MESH,
            )
            l = pltpu.make_async_remote_copy(
                src_ref=send_l.at[slot],
                dst_ref=recv_l.at[s],
                send_sem=send_sems_l.at[slot],
                recv_sem=recv_sems_l.at[s],
                device_id=(left,),
                device_id_type=pl.DeviceIdType.MESH,
            )
            r.start()
            l.start()
            rdmas_r.append(r)
            rdmas_l.append(l)

            if s < N_DEV - 2:
                nlc_r = lc_r(lax.rem(my + N_DEV - s - 2, N_DEV))
                nlc_l = lc_l(lax.rem(my + s + 2, N_DEV))
                nxt = (s + 1) % 2
                if s >= 1:
                    rdmas_r[s - 1].wait_send()
                    rdmas_l[s - 1].wait_send()
                r.wait_recv()
                l.wait_recv()
                send_r[nxt, :, :] = recv_r[s, :, :] + nlc_r
                send_l[nxt, :, :] = recv_l[s, :, :] + nlc_l
            else:
                flc_r = lc_r(my)
                flc_l = lc_l(my)
                r.wait_recv()
                l.wait_recv()
                out_ref[:, :nh] = jnp.maximum(recv_r[s, :, :] + flc_r, 0.0)
                out_ref[:, nh:] = jnp.maximum(recv_l[s, :, :] + flc_l, 0.0)

        for s in (N_DEV - 3, N_DEV - 2):
            rdmas_r[s].wait_send()
            rdmas_l[s].wait_send()

    return pl.pallas_call(
        body,
        out_shape=jax.ShapeDtypeStruct((m_per, n), jnp.float32),
        in_specs=[
            pl.BlockSpec(memory_space=pltpu.VMEM),
            pl.BlockSpec(memory_space=pltpu.VMEM),
        ],
        out_specs=pl.BlockSpec(memory_space=pltpu.VMEM),
        scratch_shapes=[
            pltpu.VMEM((2, m_per, nh), jnp.float32),
            pltpu.VMEM((2, m_per, nh), jnp.float32),
            pltpu.VMEM((N_DEV - 1, m_per, nh), jnp.float32),
            pltpu.VMEM((N_DEV - 1, m_per, nh), jnp.float32),
            pltpu.SemaphoreType.DMA((2,)),
            pltpu.SemaphoreType.DMA((2,)),
            pltpu.SemaphoreType.DMA((N_DEV - 1,)),
            pltpu.SemaphoreType.DMA((N_DEV - 1,)),
        ],
        compiler_params=pltpu.CompilerParams(collective_id=0),
    )(x, w_mat)


# device time: 90145 ns/iter; 2.0173x vs baseline; 1.1569x over previous
import jax
import jax.numpy as jnp
from jax import lax
from jax.experimental import pallas as pl
from jax.experimental.pallas import tpu as pltpu

N_DEV = 8
N_SUB = 2


def kernel(x, w_mat):
    m_global, k_shard = x.shape
    _, n = w_mat.shape
    m_per = m_global // N_DEV
    nh = n // 2
    ns = nh // N_SUB

    def body(x_ref, w_ref, out_ref,
             send_r, send_l, recv_r, recv_l,
             send_sems_r, send_sems_l, recv_sems_r, recv_sems_l):
        my = lax.axis_index("i")
        left = lax.rem(my + N_DEV - 1, N_DEV)
        right = lax.rem(my + 1, N_DEV)

        barrier = pltpu.get_barrier_semaphore()
        pl.semaphore_signal(barrier, inc=1, device_id=(left,),
                            device_id_type=pl.DeviceIdType.MESH)
        pl.semaphore_signal(barrier, inc=1, device_id=(right,),
                            device_id_type=pl.DeviceIdType.MESH)
        pl.semaphore_wait(barrier, 2)

        def lc(c, col0):
            xc = x_ref[pl.ds(c * m_per, m_per), :]
            return lax.dot_general(
                xc, w_ref[:, col0:col0 + ns],
                dimension_numbers=(((1,), (0,)), ((), ())),
                preferred_element_type=jnp.float32,
            )

        def c_r(s):
            return lax.rem(my + N_DEV - s - 1, N_DEV)

        def c_l(s):
            return lax.rem(my + s + 1, N_DEV)

        def make(direction, s, b):
            if direction == 0:
                return pltpu.make_async_remote_copy(
                    src_ref=send_r.at[s, b],
                    dst_ref=recv_r.at[s, b],
                    send_sem=send_sems_r.at[s, b],
                    recv_sem=recv_sems_r.at[s, b],
                    device_id=(right,),
                    device_id_type=pl.DeviceIdType.MESH,
                )
            return pltpu.make_async_remote_copy(
                src_ref=send_l.at[s, b],
                dst_ref=recv_l.at[s, b],
                send_sem=send_sems_l.at[s, b],
                recv_sem=recv_sems_l.at[s, b],
                device_id=(left,),
                device_id_type=pl.DeviceIdType.MESH,
            )

        rdmas = {}

        for b in range(N_SUB):
            send_r[0, b, :, :] = lc(c_r(0), b * ns)
            rdmas[(0, 0, b)] = make(0, 0, b)
            rdmas[(0, 0, b)].start()
            send_l[0, b, :, :] = lc(c_l(0), nh + b * ns)
            rdmas[(1, 0, b)] = make(1, 0, b)
            rdmas[(1, 0, b)].start()

        for s in range(N_DEV - 1):
            last = s == N_DEV - 2
            for b in range(N_SUB):
                if last:
                    nxt_r = lc(my, b * ns)
                    nxt_l = lc(my, nh + b * ns)
                else:
                    nxt_r = lc(c_r(s + 1), b * ns)
                    nxt_l = lc(c_l(s + 1), nh + b * ns)

                rdmas[(0, s, b)].wait_recv()
                if last:
                    out_ref[:, b * ns:(b + 1) * ns] = jnp.maximum(
                        recv_r[s, b, :, :] + nxt_r, 0.0)
                else:
                    send_r[s + 1, b, :, :] = recv_r[s, b, :, :] + nxt_r
                    rdmas[(0, s + 1, b)] = make(0, s + 1, b)
                    rdmas[(0, s + 1, b)].start()

                rdmas[(1, s, b)].wait_recv()
                if last:
                    out_ref[:, nh + b * ns:nh + (b + 1) * ns] = jnp.maximum(
                        recv_l[s, b, :, :] + nxt_l, 0.0)
                else:
                    send_l[s + 1, b, :, :] = recv_l[s, b, :, :] + nxt_l
                    rdmas[(1, s + 1, b)] = make(1, s + 1, b)
                    rdmas[(1, s + 1, b)].start()

        for d in range(2):
            for s in range(N_DEV - 1):
                for b in range(N_SUB):
                    rdmas[(d, s, b)].wait_send()

    nslots = N_DEV - 1
    return pl.pallas_call(
        body,
        out_shape=jax.ShapeDtypeStruct((m_per, n), jnp.float32),
        in_specs=[
            pl.BlockSpec(memory_space=pltpu.VMEM),
            pl.BlockSpec(memory_space=pltpu.VMEM),
        ],
        out_specs=pl.BlockSpec(memory_space=pltpu.VMEM),
        scratch_shapes=[
            pltpu.VMEM((nslots, N_SUB, m_per, ns), jnp.float32),
            pltpu.VMEM((nslots, N_SUB, m_per, ns), jnp.float32),
            pltpu.VMEM((nslots, N_SUB, m_per, ns), jnp.float32),
            pltpu.VMEM((nslots, N_SUB, m_per, ns), jnp.float32),
            pltpu.SemaphoreType.DMA((nslots, N_SUB)),
            pltpu.SemaphoreType.DMA((nslots, N_SUB)),
            pltpu.SemaphoreType.DMA((nslots, N_SUB)),
            pltpu.SemaphoreType.DMA((nslots, N_SUB)),
        ],
        compiler_params=pltpu.CompilerParams(collective_id=0),
    )(x, w_mat)
